# baseline (speedup 1.0000x reference)
import dataclasses
import functools

import jax
import jax.numpy as jnp
from jax import lax
from jax.experimental import pallas as pl
from jax.experimental.pallas import tpu as pltpu
from jax.experimental.pallas import tpu_sc as plsc

N = 10000
E = 320000
D = 128
G = 256

NC = 2
NS = 16
NW = NC * NS

C = 80
EPT = E // NW
CPT = EPT // C
ROWS_PER_SUB = 624
ZB = 48

BN = 1000
NB = N // BN


def _sc_aggregate(h, sd, edge_attr):
    mesh = plsc.VectorSubcoreMesh(core_axis_name="c", subcore_axis_name="s")

    @functools.partial(
        pl.kernel,
        out_type=jax.ShapeDtypeStruct((NC, N, D), jnp.float32),
        mesh=mesh,
        scratch_types=[
            pltpu.VMEM((8, 1, C), jnp.int32),
            pltpu.VMEM((C, D), jnp.float32),
            pltpu.VMEM((C, D), jnp.float32),
            pltpu.VMEM((C, D), jnp.float32),
            pltpu.VMEM((C, D), jnp.float32),
            pltpu.VMEM((ZB, D), jnp.float32),
            pltpu.VMEM_SHARED((N, D), jnp.float32),
            pltpu.SemaphoreType.DMA,
            pltpu.SemaphoreType.DMA,
            pltpu.SemaphoreType.DMA,
            pltpu.SemaphoreType.DMA,
            pltpu.SemaphoreType.DMA,
            pltpu.SemaphoreType.DMA,
            pltpu.SemaphoreType.DMA,
            pltpu.SemaphoreType.DMA,
        ],
    )
    def agg_kernel(h_hbm, sd_hbm, ea_hbm, out_hbm,
                   idxv, rows0, rows1, em0, em1, zbuf, agg,
                   isem0, isem1, isem2, isem3, dsem0, dsem1, ssem0, ssem1):
        cid = lax.axis_index("c")
        sid = lax.axis_index("s")
        wid = sid * NC + cid
        ebase = wid * EPT
        cbase = wid * CPT

        isems = (isem0, isem1, isem2, isem3)
        rows = (rows0, rows1)
        ems = (em0, em1)
        dsems = (dsem0, dsem1)
        ssems = (ssem0, ssem1)

        def fetch_idx(k, s):
            pltpu.async_copy(sd_hbm.at[pl.ds(2 * (cbase + k), 2)],
                             idxv.at[pl.ds(2 * s, 2)], isems[s])

        def wait_idx(k, s):
            pltpu.make_async_copy(sd_hbm.at[pl.ds(2 * (cbase + k), 2)],
                                  idxv.at[pl.ds(2 * s, 2)], isems[s]).wait()

        def issue_data(k, s, b):
            pltpu.async_copy(h_hbm.at[idxv.at[2 * s, 0]], rows[b], dsems[b])
            pltpu.async_copy(ea_hbm.at[pl.ds(ebase + k * C, C)],
                             ems[b], dsems[b])

        def wait_data(k, s, b):
            pltpu.make_async_copy(h_hbm.at[idxv.at[2 * s, 0]], rows[b],
                                  dsems[b]).wait()
            pltpu.make_async_copy(ea_hbm.at[pl.ds(ebase + k * C, C)],
                                  ems[b], dsems[b]).wait()

        fetch_idx(0, 0)
        fetch_idx(1, 1)
        fetch_idx(2, 2)
        wait_idx(0, 0)
        issue_data(0, 0, 0)

        @pl.loop(0, ZB)
        def _(r):
            for j in range(D // 16):
                zbuf[r, pl.ds(j * 16, 16)] = jnp.zeros((16,), jnp.float32)

        @pl.loop(0, ROWS_PER_SUB // ZB)
        def _(t):
            pltpu.sync_copy(
                zbuf.at[pl.ds(0, ZB)],
                agg.at[pl.ds(sid * ROWS_PER_SUB + t * ZB, ZB)])

        @pl.when(sid == NS - 1)
        def _():
            pltpu.sync_copy(zbuf.at[pl.ds(0, 16)],
                            agg.at[pl.ds(NS * ROWS_PER_SUB, 16)])

        plsc.subcore_barrier()

        def step(k, i):
            b = i % 2
            s = i
            s1 = (i + 1) % 4
            @pl.when((k - 1 >= 0) & (k - 1 < CPT))
            def _():
                pltpu.make_async_copy(
                    ems[1 - b], agg.at[idxv.at[2 * ((i + 3) % 4) + 1, 0]],
                    ssems[1 - b]).wait()

            @pl.when(k + 1 < CPT)
            def _():
                wait_idx(k + 1, s1)
                issue_data(k + 1, s1, 1 - b)

            @pl.when(k + 3 < CPT)
            def _():
                fetch_idx(k + 3, (i + 3) % 4)

            @pl.when(k < CPT)
            def _():
                wait_data(k, s, b)

                @plsc.parallel_loop(0, C, unroll=4)
                def _(r):
                    for j in range(D // 16):
                        sl = pl.ds(j * 16, 16)
                        m = rows[b][r, sl] + ems[b][r, sl]
                        ems[b][r, sl] = jnp.maximum(
                            m, jnp.zeros((16,), jnp.float32))

                pltpu.async_copy(ems[b], agg.at[idxv.at[2 * s + 1, 0]],
                                 ssems[b], add=True)

        @pl.loop(0, (CPT + 3) // 4 + 1)
        def _(qq):
            for i in range(4):
                step(4 * qq + i, i)

        plsc.subcore_barrier()

        pltpu.sync_copy(
            agg.at[pl.ds(sid * ROWS_PER_SUB, ROWS_PER_SUB)],
            out_hbm.at[cid, pl.ds(sid * ROWS_PER_SUB, ROWS_PER_SUB)])

        @pl.when(sid == NS - 1)
        def _():
            pltpu.sync_copy(
                agg.at[pl.ds(NS * ROWS_PER_SUB, 16)],
                out_hbm.at[cid, pl.ds(NS * ROWS_PER_SUB, 16)])

    return agg_kernel(h, sd, edge_attr)


def _tc_mlp_pool(p, h, W1, b1, W2, b2, batch3):
    def body(p0_ref, p1_ref, h_ref, W1_ref, b1_ref, W2_ref, b2_ref, bt_ref,
             hn_ref, pool_ref):
        i = pl.program_id(0)
        out = p0_ref[0] + p1_ref[0] + h_ref[...]
        hmid = jnp.maximum(
            jnp.dot(out, W1_ref[...], preferred_element_type=jnp.float32)
            + b1_ref[...], 0.0)
        hn = jnp.maximum(
            jnp.dot(hmid, W2_ref[...], preferred_element_type=jnp.float32)
            + b2_ref[...], 0.0)
        hn_ref[...] = hn

        bt = bt_ref[0, 0, :]
        onehot = (lax.broadcasted_iota(jnp.int32, (G, BN), 0)
                  == bt[None, :]).astype(jnp.float32)
        contrib = jnp.dot(onehot, hn, preferred_element_type=jnp.float32)

        @pl.when(i == 0)
        def _():
            pool_ref[...] = contrib

        @pl.when(i > 0)
        def _():
            pool_ref[...] += contrib

    return pl.pallas_call(
        body,
        grid=(NB,),
        in_specs=[
            pl.BlockSpec((1, BN, D), lambda i: (0, i, 0)),
            pl.BlockSpec((1, BN, D), lambda i: (1, i, 0)),
            pl.BlockSpec((BN, D), lambda i: (i, 0)),
            pl.BlockSpec((D, D), lambda i: (0, 0)),
            pl.BlockSpec((1, D), lambda i: (0, 0)),
            pl.BlockSpec((D, D), lambda i: (0, 0)),
            pl.BlockSpec((1, D), lambda i: (0, 0)),
            pl.BlockSpec((1, 1, BN), lambda i: (i, 0, 0)),
        ],
        out_specs=[
            pl.BlockSpec((BN, D), lambda i: (i, 0)),
            pl.BlockSpec((G, D), lambda i: (0, 0)),
        ],
        out_shape=[
            jax.ShapeDtypeStruct((N, D), jnp.float32),
            jax.ShapeDtypeStruct((G, D), jnp.float32),
        ],
    )(p, p, h, W1, b1, W2, b2, batch3)


def kernel(x, edge_index, edge_attr, batch,
           W1_0, b1_0, W2_0, b2_0,
           W1_1, b1_1, W2_1, b2_1,
           W1_2, b1_2, W2_2, b2_2):
    sd = jnp.stack([edge_index[0].reshape(E // C, C),
                    edge_index[1].reshape(E // C, C)],
                   axis=1).reshape(2 * (E // C), 1, C)
    batch3 = batch.reshape(NB, 1, BN)
    params = [(W1_0, b1_0, W2_0, b2_0),
              (W1_1, b1_1, W2_1, b2_1),
              (W1_2, b1_2, W2_2, b2_2)]
    h = x
    pools = []
    for (W1, b1, W2, b2) in params:
        p = _sc_aggregate(h, sd, edge_attr)
        h, pool = _tc_mlp_pool(p, h, W1, b1.reshape(1, D), W2,
                               b2.reshape(1, D), batch3)
        pools.append(pool)
    return jnp.concatenate(pools, axis=-1)


# speedup vs baseline: 1.0351x; 1.0351x over previous
import functools

import jax
import jax.numpy as jnp
from jax import lax
from jax.experimental import pallas as pl
from jax.experimental.pallas import tpu as pltpu
from jax.experimental.pallas import tpu_sc as plsc

N = 10000
E = 320000
D = 128
G = 256

NC = 2
NS = 16
NW = NC * NS

C = 80
EPT = E // NW
CPT = EPT // C
ROWS_PER_SUB = 624
ZB = 48

BN = 1000
NB = N // BN


def _sc_aggregate(h, src, dst, edge_attr):
    mesh = plsc.VectorSubcoreMesh(core_axis_name="c", subcore_axis_name="s")

    @functools.partial(
        pl.kernel,
        out_type=jax.ShapeDtypeStruct((NC, N, D), jnp.float32),
        mesh=mesh,
        scratch_types=[
            pltpu.VMEM((4, C), jnp.int32),
            pltpu.VMEM((4, C), jnp.int32),
            pltpu.VMEM((C, D), jnp.float32),
            pltpu.VMEM((C, D), jnp.float32),
            pltpu.VMEM((C, D), jnp.float32),
            pltpu.VMEM((C, D), jnp.float32),
            pltpu.VMEM_SHARED((N, D), jnp.float32),
            pltpu.SemaphoreType.DMA,
            pltpu.SemaphoreType.DMA,
            pltpu.SemaphoreType.DMA,
            pltpu.SemaphoreType.DMA,
            pltpu.SemaphoreType.DMA,
            pltpu.SemaphoreType.DMA,
            pltpu.SemaphoreType.DMA,
            pltpu.SemaphoreType.DMA,
        ],
    )
    def agg_kernel(h_hbm, src_hbm, dst_hbm, ea_hbm, out_hbm,
                   srcv, dstv, rows0, rows1, em0, em1, agg,
                   isem0, isem1, isem2, isem3, dsem0, dsem1, ssem0, ssem1):
        cid = lax.axis_index("c")
        sid = lax.axis_index("s")
        wid = sid * NC + cid
        ebase = wid * EPT

        isems = (isem0, isem1, isem2, isem3)
        rows = (rows0, rows1)
        ems = (em0, em1)
        dsems = (dsem0, dsem1)
        ssems = (ssem0, ssem1)

        @pl.loop(0, ZB)
        def _(r):
            for j in range(D // 16):
                em0[r, pl.ds(j * 16, 16)] = jnp.zeros((16,), jnp.float32)

        @pl.loop(0, ROWS_PER_SUB // ZB)
        def _(t):
            pltpu.sync_copy(
                em0.at[pl.ds(0, ZB)],
                agg.at[pl.ds(sid * ROWS_PER_SUB + t * ZB, ZB)])

        @pl.when(sid == NS - 1)
        def _():
            pltpu.sync_copy(em0.at[pl.ds(0, 16)],
                            agg.at[pl.ds(NS * ROWS_PER_SUB, 16)])

        plsc.subcore_barrier()

        def fetch_idx(k, s):
            pltpu.async_copy(src_hbm.at[pl.ds(ebase + k * C, C)],
                             srcv.at[s], isems[s])
            pltpu.async_copy(dst_hbm.at[pl.ds(ebase + k * C, C)],
                             dstv.at[s], isems[s])

        def wait_idx(k, s):
            pltpu.make_async_copy(src_hbm.at[pl.ds(ebase + k * C, C)],
                                  srcv.at[s], isems[s]).wait()
            pltpu.make_async_copy(dst_hbm.at[pl.ds(ebase + k * C, C)],
                                  dstv.at[s], isems[s]).wait()

        def issue_data(k, s, b):
            pltpu.async_copy(h_hbm.at[srcv.at[s]], rows[b], dsems[b])
            pltpu.async_copy(ea_hbm.at[pl.ds(ebase + k * C, C)],
                             ems[b], dsems[b])

        def wait_data(k, s, b):
            pltpu.make_async_copy(h_hbm.at[srcv.at[s]], rows[b],
                                  dsems[b]).wait()
            pltpu.make_async_copy(ea_hbm.at[pl.ds(ebase + k * C, C)],
                                  ems[b], dsems[b]).wait()

        fetch_idx(0, 0)
        fetch_idx(1, 1)
        fetch_idx(2, 2)
        wait_idx(0, 0)
        issue_data(0, 0, 0)

        def step(k, i):
            b = i % 2
            s = i
            s1 = (i + 1) % 4
            @pl.when((k - 1 >= 0) & (k - 1 < CPT))
            def _():
                pltpu.make_async_copy(
                    ems[1 - b], agg.at[dstv.at[(i + 3) % 4]],
                    ssems[1 - b]).wait()

            @pl.when(k + 1 < CPT)
            def _():
                wait_idx(k + 1, s1)
                issue_data(k + 1, s1, 1 - b)

            @pl.when(k + 3 < CPT)
            def _():
                fetch_idx(k + 3, (i + 3) % 4)

            @pl.when(k < CPT)
            def _():
                wait_data(k, s, b)

                @pl.loop(0, C)
                def _(r):
                    for j in range(D // 16):
                        sl = pl.ds(j * 16, 16)
                        m = rows[b][r, sl] + ems[b][r, sl]
                        ems[b][r, sl] = jnp.maximum(
                            m, jnp.zeros((16,), jnp.float32))

                pltpu.async_copy(ems[b], agg.at[dstv.at[s]], ssems[b],
                                 add=True)

        @pl.loop(0, (CPT + 3) // 4 + 1)
        def _(qq):
            for i in range(4):
                step(4 * qq + i, i)

        plsc.subcore_barrier()

        pltpu.sync_copy(
            agg.at[pl.ds(sid * ROWS_PER_SUB, ROWS_PER_SUB)],
            out_hbm.at[cid, pl.ds(sid * ROWS_PER_SUB, ROWS_PER_SUB)])

        @pl.when(sid == NS - 1)
        def _():
            pltpu.sync_copy(
                agg.at[pl.ds(NS * ROWS_PER_SUB, 16)],
                out_hbm.at[cid, pl.ds(NS * ROWS_PER_SUB, 16)])

    return agg_kernel(h, src, dst, edge_attr)


def _tc_mlp_pool(p, h, W1, b1, W2, b2, batch3):
    def body(p0_ref, p1_ref, h_ref, W1_ref, b1_ref, W2_ref, b2_ref, bt_ref,
             hn_ref, pool_ref):
        i = pl.program_id(0)
        out = p0_ref[0] + p1_ref[0] + h_ref[...]
        hmid = jnp.maximum(
            jnp.dot(out, W1_ref[...], preferred_element_type=jnp.float32)
            + b1_ref[...], 0.0)
        hn = jnp.maximum(
            jnp.dot(hmid, W2_ref[...], preferred_element_type=jnp.float32)
            + b2_ref[...], 0.0)
        hn_ref[...] = hn

        bt = bt_ref[0, 0, :]
        onehot = (lax.broadcasted_iota(jnp.int32, (G, BN), 0)
                  == bt[None, :]).astype(jnp.float32)
        contrib = jnp.dot(onehot, hn, preferred_element_type=jnp.float32)

        @pl.when(i == 0)
        def _():
            pool_ref[...] = contrib

        @pl.when(i > 0)
        def _():
            pool_ref[...] += contrib

    return pl.pallas_call(
        body,
        grid=(NB,),
        in_specs=[
            pl.BlockSpec((1, BN, D), lambda i: (0, i, 0)),
            pl.BlockSpec((1, BN, D), lambda i: (1, i, 0)),
            pl.BlockSpec((BN, D), lambda i: (i, 0)),
            pl.BlockSpec((D, D), lambda i: (0, 0)),
            pl.BlockSpec((1, D), lambda i: (0, 0)),
            pl.BlockSpec((D, D), lambda i: (0, 0)),
            pl.BlockSpec((1, D), lambda i: (0, 0)),
            pl.BlockSpec((1, 1, BN), lambda i: (i, 0, 0)),
        ],
        out_specs=[
            pl.BlockSpec((BN, D), lambda i: (i, 0)),
            pl.BlockSpec((G, D), lambda i: (0, 0)),
        ],
        out_shape=[
            jax.ShapeDtypeStruct((N, D), jnp.float32),
            jax.ShapeDtypeStruct((G, D), jnp.float32),
        ],
    )(p, p, h, W1, b1, W2, b2, batch3)


def kernel(x, edge_index, edge_attr, batch,
           W1_0, b1_0, W2_0, b2_0,
           W1_1, b1_1, W2_1, b2_1,
           W1_2, b1_2, W2_2, b2_2):
    src = edge_index[0]
    dst = edge_index[1]
    batch3 = batch.reshape(NB, 1, BN)
    params = [(W1_0, b1_0, W2_0, b2_0),
              (W1_1, b1_1, W2_1, b2_1),
              (W1_2, b1_2, W2_2, b2_2)]
    h = x
    pools = []
    for (W1, b1, W2, b2) in params:
        p = _sc_aggregate(h, src, dst, edge_attr)
        h, pool = _tc_mlp_pool(p, h, W1, b1.reshape(1, D), W2,
                               b2.reshape(1, D), batch3)
        pools.append(pool)
    return jnp.concatenate(pools, axis=-1)
